# Initial kernel scaffold; baseline (speedup 1.0000x reference)
import jax
import jax.numpy as jnp
from jax.experimental import pallas as pl


def kernel(x, gate_w, expert_w):
    raise NotImplementedError("write your pallas kernel here")


# speedup vs baseline: 1.2785x; 1.2785x over previous
import functools

import jax
import jax.numpy as jnp
from jax.experimental import pallas as pl
from jax.experimental.pallas import tpu as pltpu

DIM = 2048
EXPERT_DIM = 4096
N_EXPERTS = 8
TOP_K = 2
TOKENS = 4096

NEG = -1e30


def _gating_body(x_ref, gw_ref, comb_ref):
    xhi = x_ref[...].astype(jnp.bfloat16)
    ghi = gw_ref[...].astype(jnp.bfloat16)
    logits = jax.lax.dot_general(
        xhi, ghi, (((1,), (1,)), ((), ())), preferred_element_type=jnp.float32)

    iota = jax.lax.broadcasted_iota(jnp.int32, logits.shape, 1)
    m0 = jnp.max(logits, axis=1, keepdims=True)
    e0 = jnp.min(jnp.where(logits == m0, iota, N_EXPERTS), axis=1, keepdims=True)
    l2 = jnp.where(iota == e0, NEG, logits)
    m1 = jnp.max(l2, axis=1, keepdims=True)
    e1 = jnp.min(jnp.where(l2 == m1, iota, N_EXPERTS), axis=1, keepdims=True)
    t = jnp.exp(m1 - m0)
    w0 = 1.0 / (1.0 + t)
    w1 = t / (1.0 + t)
    comb_ref[...] = (jnp.where(iota == e0, w0, 0.0)
                     + jnp.where(iota == e1, w1, 0.0))


def _expert_body(x_ref, w_ref, comb_ref, out_ref):
    e = pl.program_id(2)
    xb = x_ref[...].astype(jnp.bfloat16)
    wb = w_ref[0].astype(jnp.bfloat16)
    y = jax.lax.dot_general(
        xb, wb, (((1,), (1,)), ((), ())), preferred_element_type=jnp.float32)
    iota = jax.lax.broadcasted_iota(jnp.int32, comb_ref.shape, 1)
    c = jnp.sum(jnp.where(iota == e, comb_ref[...], 0.0), axis=1, keepdims=True)

    @pl.when(e == 0)
    def _():
        out_ref[...] = c * y

    @pl.when(e != 0)
    def _():
        out_ref[...] += c * y


def kernel(x, gate_w, expert_w):
    tblk = 1024
    combine = pl.pallas_call(
        _gating_body,
        grid=(TOKENS // tblk,),
        in_specs=[
            pl.BlockSpec((tblk, DIM), lambda m: (m, 0)),
            pl.BlockSpec((N_EXPERTS, DIM), lambda m: (0, 0)),
        ],
        out_specs=pl.BlockSpec((tblk, N_EXPERTS), lambda m: (m, 0)),
        out_shape=jax.ShapeDtypeStruct((TOKENS, N_EXPERTS), jnp.float32),
    )(x, gate_w)

    mblk, nblk = 1024, 1024
    out = pl.pallas_call(
        _expert_body,
        grid=(TOKENS // mblk, EXPERT_DIM // nblk, N_EXPERTS),
        in_specs=[
            pl.BlockSpec((mblk, DIM), lambda m, n, e: (m, 0)),
            pl.BlockSpec((1, nblk, DIM), lambda m, n, e: (e, n, 0)),
            pl.BlockSpec((mblk, N_EXPERTS), lambda m, n, e: (m, 0)),
        ],
        out_specs=pl.BlockSpec((mblk, nblk), lambda m, n, e: (m, n)),
        out_shape=jax.ShapeDtypeStruct((TOKENS, EXPERT_DIM), jnp.float32),
    )(x, expert_w, combine)
    return out
